# baseline (speedup 1.0000x reference)
import jax
import jax.numpy as jnp
from jax.experimental import pallas as pl
from jax.experimental.pallas import tpu as pltpu

_H = 16
_KV = 4
_EPS = 1e-6


def _bdot(a, b):
    return jax.lax.dot_general(
        a.astype(jnp.bfloat16), b.astype(jnp.bfloat16),
        (((a.ndim - 1,), (0,)), ((), ())),
        preferred_element_type=jnp.float32)


def _qkv_rope_kernel(x_ref, cos_ref, sin_ref, ln1_ref, wqkv_ref,
                     q_ref, k_ref, v_ref):
    x = x_ref[...]
    var = jnp.mean(x * x, axis=-1, keepdims=True)
    h = x * jax.lax.rsqrt(var + _EPS) * ln1_ref[...]
    qkv = _bdot(h, wqkv_ref[...])
    bs = x.shape[0]
    hd = cos_ref.shape[-1]
    nh = qkv.shape[-1] // hd
    qkv = qkv.reshape(bs, nh, hd)
    c = cos_ref[...][:, None, :]
    s = sin_ref[...][:, None, :]
    qk = qkv[:, :_H + _KV, :]
    x1 = qk[..., :hd // 2]
    x2 = qk[..., hd // 2:]
    rot = jnp.concatenate([-x2, x1], axis=-1)
    qk = qk * c + rot * s
    q_ref[...] = qk[:, :_H, :].transpose(1, 0, 2).astype(jnp.bfloat16)
    k_ref[...] = qk[:, _H:, :].transpose(1, 0, 2).astype(jnp.bfloat16)
    v_ref[...] = qkv[:, _H + _KV:, :].transpose(1, 0, 2).astype(jnp.bfloat16)


def _attn_kernel(q_ref, k_ref, v_ref, o_ref, *, bq, s_len):
    i = pl.program_id(1)
    q = q_ref[0]
    k = k_ref[0]
    hd = q.shape[-1]
    s = jax.lax.dot_general(
        q.astype(jnp.bfloat16), k.astype(jnp.bfloat16),
        (((1,), (1,)), ((), ())), preferred_element_type=jnp.float32)
    s = s * (1.0 / jnp.sqrt(jnp.float32(hd)))
    rows = i * bq + jax.lax.broadcasted_iota(jnp.int32, (bq, s_len), 0)
    cols = jax.lax.broadcasted_iota(jnp.int32, (bq, s_len), 1)
    s = jnp.where(cols <= rows, s, jnp.finfo(jnp.float32).min)
    m = jnp.max(s, axis=-1, keepdims=True)
    p = jnp.exp(s - m)
    den = jnp.sum(p, axis=-1, keepdims=True)
    o_ref[0] = _bdot(p, v_ref[0]) / den


def _oproj_router_kernel(attn_ref, hid_ref, ln2_ref, wo_ref, gate_ref,
                         res_ref, x2_ref, logits_ref):
    a = attn_ref[...]
    r = hid_ref[...] + _bdot(a, wo_ref[...])
    res_ref[...] = r
    var = jnp.mean(r * r, axis=-1, keepdims=True)
    x2 = r * jax.lax.rsqrt(var + _EPS) * ln2_ref[...]
    x2_ref[...] = x2
    logits_ref[...] = _bdot(x2, gate_ref[...])


def _moe_g_kernel(be_ref, nv_ref, xs_ref, w1_ref, w3_ref, g_ref):
    i = pl.program_id(1)

    @pl.when(i < nv_ref[0])
    def _():
        x = xs_ref[...]
        g = jax.lax.dot_general(x, w1_ref[0], (((1,), (0,)), ((), ())),
                                preferred_element_type=jnp.float32)
        u = jax.lax.dot_general(x, w3_ref[0], (((1,), (0,)), ((), ())),
                                preferred_element_type=jnp.float32)
        g = g * jax.lax.logistic(g) * u
        g_ref[...] = g.astype(jnp.bfloat16)


def _moe_y_kernel(be_ref, nv_ref, g_ref, w2_ref, y_ref):
    i = pl.program_id(0)

    @pl.when(i < nv_ref[0])
    def _():
        y_ref[...] = jax.lax.dot_general(
            g_ref[...], w2_ref[0], (((1,), (0,)), ((), ())),
            preferred_element_type=jnp.float32)


def kernel(hidden_states, cos, sin, ln1_w, w_qkv, w_o, ln2_w, gate_w,
           w1, w3, w2):
    b, s_len, d = hidden_states.shape
    hd = cos.shape[-1]
    e = gate_w.shape[-1]
    f = w1.shape[-1]
    topk = 2
    rep = _H // _KV

    x = hidden_states.reshape(s_len, d)
    ln1 = ln1_w.reshape(1, d)
    ln2 = ln2_w.reshape(1, d)

    bs = min(256, s_len)
    nq = s_len // bs

    q, k, v = pl.pallas_call(
        _qkv_rope_kernel,
        grid=(nq,),
        in_specs=[
            pl.BlockSpec((bs, d), lambda i: (i, 0)),
            pl.BlockSpec((bs, hd), lambda i: (i, 0)),
            pl.BlockSpec((bs, hd), lambda i: (i, 0)),
            pl.BlockSpec((1, d), lambda i: (0, 0)),
            pl.BlockSpec((d, (_H + 2 * _KV) * hd), lambda i: (0, 0)),
        ],
        out_specs=[
            pl.BlockSpec((_H, bs, hd), lambda i: (0, i, 0)),
            pl.BlockSpec((_KV, bs, hd), lambda i: (0, i, 0)),
            pl.BlockSpec((_KV, bs, hd), lambda i: (0, i, 0)),
        ],
        out_shape=[
            jax.ShapeDtypeStruct((_H, s_len, hd), jnp.bfloat16),
            jax.ShapeDtypeStruct((_KV, s_len, hd), jnp.bfloat16),
            jax.ShapeDtypeStruct((_KV, s_len, hd), jnp.bfloat16),
        ],
    )(x, cos, sin, ln1, w_qkv)

    import functools
    attn = pl.pallas_call(
        functools.partial(_attn_kernel, bq=bs, s_len=s_len),
        grid=(_H, nq),
        in_specs=[
            pl.BlockSpec((1, bs, hd), lambda h, i: (h, i, 0)),
            pl.BlockSpec((1, s_len, hd), lambda h, i: (h // rep, 0, 0)),
            pl.BlockSpec((1, s_len, hd), lambda h, i: (h // rep, 0, 0)),
        ],
        out_specs=pl.BlockSpec((1, bs, hd), lambda h, i: (h, i, 0)),
        out_shape=jax.ShapeDtypeStruct((_H, s_len, hd), jnp.float32),
    )(q, k, v)

    attn_t = attn.transpose(1, 0, 2).reshape(s_len, _H * hd)

    res, x2, logits = pl.pallas_call(
        _oproj_router_kernel,
        grid=(nq,),
        in_specs=[
            pl.BlockSpec((bs, _H * hd), lambda i: (i, 0)),
            pl.BlockSpec((bs, d), lambda i: (i, 0)),
            pl.BlockSpec((1, d), lambda i: (0, 0)),
            pl.BlockSpec((_H * hd, d), lambda i: (0, 0)),
            pl.BlockSpec((d, e), lambda i: (0, 0)),
        ],
        out_specs=[
            pl.BlockSpec((bs, d), lambda i: (i, 0)),
            pl.BlockSpec((bs, d), lambda i: (i, 0)),
            pl.BlockSpec((bs, e), lambda i: (i, 0)),
        ],
        out_shape=[
            jax.ShapeDtypeStruct((s_len, d), jnp.float32),
            jax.ShapeDtypeStruct((s_len, d), jnp.float32),
            jax.ShapeDtypeStruct((s_len, e), jnp.float32),
        ],
    )(attn_t, x, ln2, w_o, gate_w)

    t = s_len
    p_cnt = t * topk
    bm = 256
    padt = p_cnt + e * bm
    nb = padt // bm
    bf = f // 4
    nf = 4

    rprobs = jax.nn.softmax(logits, axis=-1)
    topv, topi = jax.lax.top_k(rprobs, topk)
    gates = topv / jnp.sum(topv, axis=-1, keepdims=True)

    eid = topi.reshape(-1).astype(jnp.int32)
    tok = jnp.arange(p_cnt, dtype=jnp.int32) // topk
    order = jnp.argsort(eid, stable=True)
    seid = eid[order]
    stok = tok[order]

    counts = jnp.sum(jax.nn.one_hot(eid, e, dtype=jnp.int32), axis=0)
    cstart = jnp.concatenate([jnp.zeros(1, jnp.int32),
                              jnp.cumsum(counts)[:-1]])
    nblk = (counts + bm - 1) // bm
    cumblk = jnp.cumsum(nblk)
    bstart = jnp.concatenate([jnp.zeros(1, jnp.int32), cumblk[:-1]])
    nvalid = cumblk[-1:].astype(jnp.int32)

    dest = bstart[seid] * bm + (jnp.arange(p_cnt, dtype=jnp.int32)
                                - cstart[seid])
    rows_tok = jnp.zeros(padt, jnp.int32).at[dest].set(stok)
    pos_of_pair = jnp.zeros(p_cnt, jnp.int32).at[order].set(dest)
    token_pos = pos_of_pair.reshape(t, topk)
    block_expert = jnp.minimum(
        jnp.searchsorted(cumblk, jnp.arange(nb, dtype=jnp.int32),
                         side='right'),
        e - 1).astype(jnp.int32)

    xs = jnp.take(x2.astype(jnp.bfloat16), rows_tok, axis=0)
    w1b = w1.astype(jnp.bfloat16)
    w3b = w3.astype(jnp.bfloat16)
    w2b = w2.astype(jnp.bfloat16)

    g = pl.pallas_call(
        _moe_g_kernel,
        grid_spec=pltpu.PrefetchScalarGridSpec(
            num_scalar_prefetch=2,
            grid=(nf, nb),
            in_specs=[
                pl.BlockSpec((bm, d), lambda j, i, be, nv: (i, 0)),
                pl.BlockSpec((1, d, bf), lambda j, i, be, nv: (be[i], 0, j)),
                pl.BlockSpec((1, d, bf), lambda j, i, be, nv: (be[i], 0, j)),
            ],
            out_specs=pl.BlockSpec((bm, bf), lambda j, i, be, nv: (i, j)),
        ),
        out_shape=jax.ShapeDtypeStruct((padt, f), jnp.bfloat16),
        compiler_params=pltpu.CompilerParams(
            dimension_semantics=("arbitrary", "arbitrary"),
        ),
    )(block_expert, nvalid, xs, w1b, w3b)

    ys = pl.pallas_call(
        _moe_y_kernel,
        grid_spec=pltpu.PrefetchScalarGridSpec(
            num_scalar_prefetch=2,
            grid=(nb,),
            in_specs=[
                pl.BlockSpec((bm, f), lambda i, be, nv: (i, 0)),
                pl.BlockSpec((1, f, d), lambda i, be, nv: (be[i], 0, 0)),
            ],
            out_specs=pl.BlockSpec((bm, d), lambda i, be, nv: (i, 0)),
        ),
        out_shape=jax.ShapeDtypeStruct((padt, d), jnp.float32),
        compiler_params=pltpu.CompilerParams(
            dimension_semantics=("arbitrary",),
        ),
    )(block_expert, nvalid, g, w2b)

    out = (gates[:, 0, None] * jnp.take(ys, token_pos[:, 0], axis=0)
           + gates[:, 1, None] * jnp.take(ys, token_pos[:, 1], axis=0))

    return (out.reshape(b, s_len, d), res.reshape(b, s_len, d))


# speedup vs baseline: 1.1531x; 1.1531x over previous
import jax
import jax.numpy as jnp
from jax.experimental import pallas as pl
from jax.experimental.pallas import tpu as pltpu

_H = 16
_KV = 4
_EPS = 1e-6


def _bdot(a, b):
    return jax.lax.dot_general(
        a.astype(jnp.bfloat16), b.astype(jnp.bfloat16),
        (((a.ndim - 1,), (0,)), ((), ())),
        preferred_element_type=jnp.float32)


def _qkv_rope_kernel(x_ref, cos_ref, sin_ref, ln1_ref, wqkv_ref,
                     q_ref, k_ref, v_ref):
    x = x_ref[...]
    var = jnp.mean(x * x, axis=-1, keepdims=True)
    h = x * jax.lax.rsqrt(var + _EPS) * ln1_ref[...]
    qkv = _bdot(h, wqkv_ref[...])
    bs = x.shape[0]
    hd = cos_ref.shape[-1]
    nh = qkv.shape[-1] // hd
    qkv = qkv.reshape(bs, nh, hd)
    c = cos_ref[...][:, None, :]
    s = sin_ref[...][:, None, :]
    qk = qkv[:, :_H + _KV, :]
    x1 = qk[..., :hd // 2]
    x2 = qk[..., hd // 2:]
    rot = jnp.concatenate([-x2, x1], axis=-1)
    qk = qk * c + rot * s
    q_ref[...] = qk[:, :_H, :].transpose(1, 0, 2).astype(jnp.bfloat16)
    k_ref[...] = qk[:, _H:, :].transpose(1, 0, 2).astype(jnp.bfloat16)
    v_ref[...] = qkv[:, _H + _KV:, :].transpose(1, 0, 2).astype(jnp.bfloat16)


def _attn_kernel(q_ref, k_ref, v_ref, o_ref, *, bq, s_len):
    i = pl.program_id(1)
    q = q_ref[0]
    k = k_ref[0]
    hd = q.shape[-1]
    s = jax.lax.dot_general(
        q.astype(jnp.bfloat16), k.astype(jnp.bfloat16),
        (((1,), (1,)), ((), ())), preferred_element_type=jnp.float32)
    s = s * (1.0 / jnp.sqrt(jnp.float32(hd)))
    rows = i * bq + jax.lax.broadcasted_iota(jnp.int32, (bq, s_len), 0)
    cols = jax.lax.broadcasted_iota(jnp.int32, (bq, s_len), 1)
    s = jnp.where(cols <= rows, s, jnp.finfo(jnp.float32).min)
    m = jnp.max(s, axis=-1, keepdims=True)
    p = jnp.exp(s - m)
    den = jnp.sum(p, axis=-1, keepdims=True)
    o_ref[0] = _bdot(p, v_ref[0]) / den


def _oproj_router_kernel(attn_ref, hid_ref, ln2_ref, wo_ref, gate_ref,
                         res_ref, x2_ref, logits_ref):
    a = attn_ref[...]
    r = hid_ref[...] + _bdot(a, wo_ref[...])
    res_ref[...] = r
    var = jnp.mean(r * r, axis=-1, keepdims=True)
    x2 = r * jax.lax.rsqrt(var + _EPS) * ln2_ref[...]
    x2_ref[...] = x2
    logits_ref[...] = _bdot(x2, gate_ref[...])


def _moe_g_kernel(be_ref, nv_ref, xs_ref, w1_ref, w3_ref, g_ref):
    i = pl.program_id(1)

    @pl.when(i < nv_ref[0])
    def _():
        x = xs_ref[...]
        g = _bdot(x, w1_ref[0])
        u = _bdot(x, w3_ref[0])
        g = g * jax.lax.logistic(g) * u
        g_ref[...] = g.astype(jnp.bfloat16)


def _moe_y_kernel(be_ref, nv_ref, g_ref, w2_ref, y_ref):
    i = pl.program_id(0)

    @pl.when(i < nv_ref[0])
    def _():
        y_ref[...] = _bdot(g_ref[...], w2_ref[0])


def kernel(hidden_states, cos, sin, ln1_w, w_qkv, w_o, ln2_w, gate_w,
           w1, w3, w2):
    b, s_len, d = hidden_states.shape
    hd = cos.shape[-1]
    e = gate_w.shape[-1]
    f = w1.shape[-1]
    topk = 2
    rep = _H // _KV

    x = hidden_states.reshape(s_len, d)
    ln1 = ln1_w.reshape(1, d)
    ln2 = ln2_w.reshape(1, d)

    bs = min(256, s_len)
    nq = s_len // bs

    q, k, v = pl.pallas_call(
        _qkv_rope_kernel,
        grid=(nq,),
        in_specs=[
            pl.BlockSpec((bs, d), lambda i: (i, 0)),
            pl.BlockSpec((bs, hd), lambda i: (i, 0)),
            pl.BlockSpec((bs, hd), lambda i: (i, 0)),
            pl.BlockSpec((1, d), lambda i: (0, 0)),
            pl.BlockSpec((d, (_H + 2 * _KV) * hd), lambda i: (0, 0)),
        ],
        out_specs=[
            pl.BlockSpec((_H, bs, hd), lambda i: (0, i, 0)),
            pl.BlockSpec((_KV, bs, hd), lambda i: (0, i, 0)),
            pl.BlockSpec((_KV, bs, hd), lambda i: (0, i, 0)),
        ],
        out_shape=[
            jax.ShapeDtypeStruct((_H, s_len, hd), jnp.bfloat16),
            jax.ShapeDtypeStruct((_KV, s_len, hd), jnp.bfloat16),
            jax.ShapeDtypeStruct((_KV, s_len, hd), jnp.bfloat16),
        ],
    )(x, cos, sin, ln1, w_qkv)

    import functools
    attn = pl.pallas_call(
        functools.partial(_attn_kernel, bq=bs, s_len=s_len),
        grid=(_H, nq),
        in_specs=[
            pl.BlockSpec((1, bs, hd), lambda h, i: (h, i, 0)),
            pl.BlockSpec((1, s_len, hd), lambda h, i: (h // rep, 0, 0)),
            pl.BlockSpec((1, s_len, hd), lambda h, i: (h // rep, 0, 0)),
        ],
        out_specs=pl.BlockSpec((1, bs, hd), lambda h, i: (h, i, 0)),
        out_shape=jax.ShapeDtypeStruct((_H, s_len, hd), jnp.float32),
    )(q, k, v)

    attn_t = attn.transpose(1, 0, 2).reshape(s_len, _H * hd)

    res, x2, logits = pl.pallas_call(
        _oproj_router_kernel,
        grid=(nq,),
        in_specs=[
            pl.BlockSpec((bs, _H * hd), lambda i: (i, 0)),
            pl.BlockSpec((bs, d), lambda i: (i, 0)),
            pl.BlockSpec((1, d), lambda i: (0, 0)),
            pl.BlockSpec((_H * hd, d), lambda i: (0, 0)),
            pl.BlockSpec((d, e), lambda i: (0, 0)),
        ],
        out_specs=[
            pl.BlockSpec((bs, d), lambda i: (i, 0)),
            pl.BlockSpec((bs, d), lambda i: (i, 0)),
            pl.BlockSpec((bs, e), lambda i: (i, 0)),
        ],
        out_shape=[
            jax.ShapeDtypeStruct((s_len, d), jnp.float32),
            jax.ShapeDtypeStruct((s_len, d), jnp.float32),
            jax.ShapeDtypeStruct((s_len, e), jnp.float32),
        ],
    )(attn_t, x, ln2, w_o, gate_w)

    t = s_len
    p_cnt = t * topk
    bm = 256
    padt = p_cnt + e * bm
    nb = padt // bm
    bf = f // 4
    nf = 4

    rprobs = jax.nn.softmax(logits, axis=-1)
    topv, topi = jax.lax.top_k(rprobs, topk)
    gates = topv / jnp.sum(topv, axis=-1, keepdims=True)

    eid = topi.reshape(-1).astype(jnp.int32)
    tok = jnp.arange(p_cnt, dtype=jnp.int32) // topk
    order = jnp.argsort(eid, stable=True)
    seid = eid[order]
    stok = tok[order]

    counts = jnp.sum(jax.nn.one_hot(eid, e, dtype=jnp.int32), axis=0)
    cstart = jnp.concatenate([jnp.zeros(1, jnp.int32),
                              jnp.cumsum(counts)[:-1]])
    nblk = (counts + bm - 1) // bm
    cumblk = jnp.cumsum(nblk)
    bstart = jnp.concatenate([jnp.zeros(1, jnp.int32), cumblk[:-1]])
    nvalid = cumblk[-1:].astype(jnp.int32)

    dest = bstart[seid] * bm + (jnp.arange(p_cnt, dtype=jnp.int32)
                                - cstart[seid])
    rows_tok = jnp.zeros(padt, jnp.int32).at[dest].set(stok)
    pos_of_pair = jnp.zeros(p_cnt, jnp.int32).at[order].set(dest)
    token_pos = pos_of_pair.reshape(t, topk)
    block_expert = jnp.minimum(
        jnp.searchsorted(cumblk, jnp.arange(nb, dtype=jnp.int32),
                         side='right'),
        e - 1).astype(jnp.int32)

    xs = jnp.take(x2.astype(jnp.bfloat16), rows_tok, axis=0)

    g = pl.pallas_call(
        _moe_g_kernel,
        grid_spec=pltpu.PrefetchScalarGridSpec(
            num_scalar_prefetch=2,
            grid=(nf, nb),
            in_specs=[
                pl.BlockSpec((bm, d), lambda j, i, be, nv: (i, 0)),
                pl.BlockSpec((1, d, bf), lambda j, i, be, nv: (be[i], 0, j)),
                pl.BlockSpec((1, d, bf), lambda j, i, be, nv: (be[i], 0, j)),
            ],
            out_specs=pl.BlockSpec((bm, bf), lambda j, i, be, nv: (i, j)),
        ),
        out_shape=jax.ShapeDtypeStruct((padt, f), jnp.bfloat16),
        compiler_params=pltpu.CompilerParams(
            dimension_semantics=("arbitrary", "arbitrary"),
        ),
    )(block_expert, nvalid, xs, w1, w3)

    ys = pl.pallas_call(
        _moe_y_kernel,
        grid_spec=pltpu.PrefetchScalarGridSpec(
            num_scalar_prefetch=2,
            grid=(nb,),
            in_specs=[
                pl.BlockSpec((bm, f), lambda i, be, nv: (i, 0)),
                pl.BlockSpec((1, f, d), lambda i, be, nv: (be[i], 0, 0)),
            ],
            out_specs=pl.BlockSpec((bm, d), lambda i, be, nv: (i, 0)),
        ),
        out_shape=jax.ShapeDtypeStruct((padt, d), jnp.float32),
        compiler_params=pltpu.CompilerParams(
            dimension_semantics=("arbitrary",),
        ),
    )(block_expert, nvalid, g, w2)

    out = (gates[:, 0, None] * jnp.take(ys, token_pos[:, 0], axis=0)
           + gates[:, 1, None] * jnp.take(ys, token_pos[:, 1], axis=0))

    return (out.reshape(b, s_len, d), res.reshape(b, s_len, d))
